# Initial kernel scaffold; baseline (speedup 1.0000x reference)
import jax
import jax.numpy as jnp
from jax.experimental import pallas as pl


def kernel(x, w_ih, w_hh, b_ih, b_hh, w_fc, b_fc):
    raise NotImplementedError("write your pallas kernel here")


# speedup vs baseline: 3.7605x; 3.7605x over previous
import jax
import jax.numpy as jnp
from jax.experimental import pallas as pl
from jax.experimental.pallas import tpu as pltpu


def _lstm_head_kernel(x_ref, wih_ref, whh_ref, b_ref, wfc_ref, bfc_ref, y_ref):
    B, T, I = x_ref.shape
    H = whh_ref.shape[0]

    xt = jnp.concatenate([x_ref[:, t, :] for t in range(T)], axis=0)
    gx = jnp.dot(xt.astype(jnp.bfloat16), wih_ref[...],
                 preferred_element_type=jnp.float32) + b_ref[...]

    whh = whh_ref[...]
    h = jnp.zeros((B, H), jnp.float32)
    c = jnp.zeros((B, H), jnp.float32)
    hs = []
    for t in range(T):
        gates = gx[t * B:(t + 1) * B, :] + jnp.dot(
            h.astype(jnp.bfloat16), whh,
            preferred_element_type=jnp.float32)
        i_g = jax.nn.sigmoid(gates[:, 0 * H:1 * H])
        f_g = jax.nn.sigmoid(gates[:, 1 * H:2 * H])
        g_g = jnp.tanh(gates[:, 2 * H:3 * H])
        o_g = jax.nn.sigmoid(gates[:, 3 * H:4 * H])
        c = f_g * c + i_g * g_g
        h = o_g * jnp.tanh(c)
        hs.append(h)

    hst = jnp.concatenate(hs, axis=0)
    y = (jnp.dot(hst, wfc_ref[...], preferred_element_type=jnp.float32)
         + bfc_ref[...]).astype(y_ref.dtype)
    for t in range(T):
        y_ref[:, t, :] = y[t * B:(t + 1) * B, :]


def kernel(x, w_ih, w_hh, b_ih, b_hh, w_fc, b_fc):
    B, T, I = x.shape
    H = w_hh.shape[1]
    O = w_fc.shape[0]

    wih_t = w_ih.T.astype(jnp.bfloat16)
    whh_t = w_hh.T.astype(jnp.bfloat16)
    b = (b_ih + b_hh).reshape(1, 4 * H)
    wfc_t = w_fc.T
    bfc = b_fc.reshape(1, O)

    return pl.pallas_call(
        _lstm_head_kernel,
        out_shape=jax.ShapeDtypeStruct((B, T, O), x.dtype),
        grid_spec=pltpu.PrefetchScalarGridSpec(
            num_scalar_prefetch=0,
            grid=(1,),
            in_specs=[
                pl.BlockSpec((B, T, I), lambda i: (0, 0, 0)),
                pl.BlockSpec((I, 4 * H), lambda i: (0, 0)),
                pl.BlockSpec((H, 4 * H), lambda i: (0, 0)),
                pl.BlockSpec((1, 4 * H), lambda i: (0, 0)),
                pl.BlockSpec((H, O), lambda i: (0, 0)),
                pl.BlockSpec((1, O), lambda i: (0, 0)),
            ],
            out_specs=pl.BlockSpec((B, T, O), lambda i: (0, 0, 0)),
        ),
        compiler_params=pltpu.CompilerParams(
            dimension_semantics=("arbitrary",)),
    )(x, wih_t, whh_t, b, wfc_t, bfc)


# speedup vs baseline: 6.6917x; 1.7795x over previous
import jax
import jax.numpy as jnp
from jax.experimental import pallas as pl
from jax.experimental.pallas import tpu as pltpu


def _lstm_head_kernel(x_ref, wih_ref, whh_ref, bih_ref, bhh_ref, wfc_ref,
                      bfc_ref, y_ref, whht_ref):
    B, T, I = x_ref.shape
    H = whh_ref.shape[1]

    whht_ref[...] = jnp.transpose(whh_ref[...].astype(jnp.bfloat16))
    wih = jnp.transpose(wih_ref[...].astype(jnp.bfloat16))
    whh = whht_ref[...]
    wfc = jnp.transpose(wfc_ref[...])
    bias = bih_ref[...] + bhh_ref[...]

    xt = jnp.concatenate([x_ref[:, t, :] for t in range(T)], axis=0)
    gx = jnp.dot(xt.astype(jnp.bfloat16), wih,
                 preferred_element_type=jnp.float32) + bias

    h = jnp.zeros((B, H), jnp.float32)
    c = jnp.zeros((B, H), jnp.float32)
    hs = []
    for t in range(T):
        gates = gx[t * B:(t + 1) * B, :] + jnp.dot(
            h.astype(jnp.bfloat16), whh,
            preferred_element_type=jnp.float32)
        i_g = jax.nn.sigmoid(gates[:, 0 * H:1 * H])
        f_g = jax.nn.sigmoid(gates[:, 1 * H:2 * H])
        g_g = jnp.tanh(gates[:, 2 * H:3 * H])
        o_g = jax.nn.sigmoid(gates[:, 3 * H:4 * H])
        c = f_g * c + i_g * g_g
        h = o_g * jnp.tanh(c)
        hs.append(h)

    hst = jnp.concatenate(hs, axis=0)
    y = (jnp.dot(hst, wfc, preferred_element_type=jnp.float32)
         + bfc_ref[...]).astype(y_ref.dtype)
    for t in range(T):
        y_ref[:, t, :] = y[t * B:(t + 1) * B, :]


def kernel(x, w_ih, w_hh, b_ih, b_hh, w_fc, b_fc):
    B, T, I = x.shape
    H = w_hh.shape[1]
    O = w_fc.shape[0]

    bih = b_ih.reshape(1, 4 * H)
    bhh = b_hh.reshape(1, 4 * H)
    bfc = b_fc.reshape(1, O)

    return pl.pallas_call(
        _lstm_head_kernel,
        out_shape=jax.ShapeDtypeStruct((B, T, O), x.dtype),
        grid_spec=pltpu.PrefetchScalarGridSpec(
            num_scalar_prefetch=0,
            grid=(1,),
            in_specs=[
                pl.BlockSpec((B, T, I), lambda i: (0, 0, 0)),
                pl.BlockSpec((4 * H, I), lambda i: (0, 0)),
                pl.BlockSpec((4 * H, H), lambda i: (0, 0)),
                pl.BlockSpec((1, 4 * H), lambda i: (0, 0)),
                pl.BlockSpec((1, 4 * H), lambda i: (0, 0)),
                pl.BlockSpec((O, H), lambda i: (0, 0)),
                pl.BlockSpec((1, O), lambda i: (0, 0)),
            ],
            out_specs=pl.BlockSpec((B, T, O), lambda i: (0, 0, 0)),
            scratch_shapes=[pltpu.VMEM((H, 4 * H), jnp.bfloat16)],
        ),
        compiler_params=pltpu.CompilerParams(
            dimension_semantics=("arbitrary",)),
    )(x, w_ih, w_hh, bih, bhh, w_fc, bfc)


# speedup vs baseline: 6.8187x; 1.0190x over previous
import jax
import jax.numpy as jnp
from jax.experimental import pallas as pl
from jax.experimental.pallas import tpu as pltpu


def _lstm_head_kernel(x_ref, wih_hbm, whh_hbm, bih_ref, bhh_ref, wfc_hbm,
                      bfc_ref, y_ref, wih_v, whh_v, wfc_v, whht_ref, sems):
    B, T, I = x_ref.shape
    H = whh_hbm.shape[1]

    cp_wih = pltpu.make_async_copy(wih_hbm, wih_v, sems.at[0])
    cp_whh = pltpu.make_async_copy(whh_hbm, whh_v, sems.at[1])
    cp_wfc = pltpu.make_async_copy(wfc_hbm, wfc_v, sems.at[2])
    cp_wih.start()
    cp_whh.start()
    cp_wfc.start()

    bias = bih_ref[...] + bhh_ref[...]
    xt = jnp.concatenate([x_ref[:, t, :] for t in range(T)], axis=0)
    xb = xt.astype(jnp.bfloat16)

    cp_wih.wait()
    wih = jnp.transpose(wih_v[...].astype(jnp.bfloat16))
    gx = jnp.dot(xb, wih, preferred_element_type=jnp.float32) + bias

    cp_whh.wait()
    whht_ref[...] = jnp.transpose(whh_v[...].astype(jnp.bfloat16))
    whh = whht_ref[...]

    h = jnp.zeros((B, H), jnp.float32)
    c = jnp.zeros((B, H), jnp.float32)
    hs = []
    for t in range(T):
        gates = gx[t * B:(t + 1) * B, :] + jnp.dot(
            h.astype(jnp.bfloat16), whh,
            preferred_element_type=jnp.float32)
        i_g = jax.nn.sigmoid(gates[:, 0 * H:1 * H])
        f_g = jax.nn.sigmoid(gates[:, 1 * H:2 * H])
        g_g = jnp.tanh(gates[:, 2 * H:3 * H])
        o_g = jax.nn.sigmoid(gates[:, 3 * H:4 * H])
        c = f_g * c + i_g * g_g
        h = o_g * jnp.tanh(c)
        hs.append(h)

    hst = jnp.concatenate(hs, axis=0)
    cp_wfc.wait()
    wfc = jnp.transpose(wfc_v[...])
    y = (jnp.dot(hst, wfc, preferred_element_type=jnp.float32)
         + bfc_ref[...]).astype(y_ref.dtype)
    for t in range(T):
        y_ref[:, t, :] = y[t * B:(t + 1) * B, :]


def kernel(x, w_ih, w_hh, b_ih, b_hh, w_fc, b_fc):
    B, T, I = x.shape
    H = w_hh.shape[1]
    O = w_fc.shape[0]

    bih = b_ih.reshape(1, 4 * H)
    bhh = b_hh.reshape(1, 4 * H)
    bfc = b_fc.reshape(1, O)

    return pl.pallas_call(
        _lstm_head_kernel,
        out_shape=jax.ShapeDtypeStruct((B, T, O), x.dtype),
        in_specs=[
            pl.BlockSpec(memory_space=pltpu.VMEM),
            pl.BlockSpec(memory_space=pl.ANY),
            pl.BlockSpec(memory_space=pl.ANY),
            pl.BlockSpec(memory_space=pltpu.VMEM),
            pl.BlockSpec(memory_space=pltpu.VMEM),
            pl.BlockSpec(memory_space=pl.ANY),
            pl.BlockSpec(memory_space=pltpu.VMEM),
        ],
        out_specs=pl.BlockSpec(memory_space=pltpu.VMEM),
        scratch_shapes=[
            pltpu.VMEM((4 * H, I), jnp.float32),
            pltpu.VMEM((4 * H, H), jnp.float32),
            pltpu.VMEM((O, H), jnp.float32),
            pltpu.VMEM((H, 4 * H), jnp.bfloat16),
            pltpu.SemaphoreType.DMA((3,)),
        ],
        compiler_params=pltpu.CompilerParams(
            vmem_limit_bytes=100 * 1024 * 1024),
    )(x, w_ih, w_hh, bih, bhh, w_fc, bfc)
